# Initial kernel scaffold; baseline (speedup 1.0000x reference)
import jax
import jax.numpy as jnp
from jax.experimental import pallas as pl


def kernel(h, x, edge_attr, We1, be1, We2, be2, Wc1, bc1, Wc2, Wn1, bn1, Wn2, bn2, ln_g, ln_b, edge_index):
    raise NotImplementedError("write your pallas kernel here")


# speedup vs baseline: 3.7639x; 3.7639x over previous
import functools

import jax
import jax.numpy as jnp
from jax import lax
from jax.experimental import pallas as pl
from jax.experimental.pallas import tpu as pltpu
from jax.experimental.pallas import tpu_sc as plsc

F32 = jnp.float32
XP = 16
CHUNK = 80
NCORE = 2
NSUB = 16
TILES = NCORE * NSUB


def _silu(v):
    return v * jax.nn.sigmoid(v)


def _prep_body(h_ref, wrt_ref, wct_ref, hr_ref, hc_ref):
    h = h_ref[...]
    hr_ref[...] = jnp.dot(h, wrt_ref[...], preferred_element_type=F32)
    hc_ref[...] = jnp.dot(h, wct_ref[...], preferred_element_type=F32)


def _prep(h, wrt, wct, bn):
    n, d = h.shape
    grid = n // bn
    return pl.pallas_call(
        _prep_body,
        grid=(grid,),
        in_specs=[
            pl.BlockSpec((bn, d), lambda i: (i, 0)),
            pl.BlockSpec((d, d), lambda i: (0, 0)),
            pl.BlockSpec((d, d), lambda i: (0, 0)),
        ],
        out_specs=[
            pl.BlockSpec((bn, d), lambda i: (i, 0)),
            pl.BlockSpec((bn, d), lambda i: (i, 0)),
        ],
        out_shape=[
            jax.ShapeDtypeStruct((n, d), F32),
            jax.ShapeDtypeStruct((n, d), F32),
        ],
    )(h, wrt, wct)


def _sc_cd_body(nch, x0_hbm, x1_hbm, x2_hbm, row_hbm, col_hbm, cd_hbm,
                idxr, idxc, bufx, x0v, x1v, x2v):
    cid = lax.axis_index("c")
    sid = lax.axis_index("s")
    wid = sid * NCORE + cid
    ebase = wid * nch * CHUNK
    pltpu.sync_copy(row_hbm.at[wid], idxr)
    pltpu.sync_copy(col_hbm.at[wid], idxc)
    pltpu.sync_copy(x0_hbm, x0v)
    pltpu.sync_copy(x1_hbm, x1v)
    pltpu.sync_copy(x2_hbm, x2v)

    def step(j, carry):
        for grp in range(CHUNK // 16):
            sl = pl.ds(grp * 16, 16)
            ir = idxr[j, sl]
            ic = idxc[j, sl]
            irh, irl = ir >> 7, ir & 127
            ich, icl = ic >> 7, ic & 127
            dx = (plsc.load_gather(x0v, [irh, irl])
                  - plsc.load_gather(x0v, [ich, icl]))
            dy = (plsc.load_gather(x1v, [irh, irl])
                  - plsc.load_gather(x1v, [ich, icl]))
            dz = (plsc.load_gather(x2v, [irh, irl])
                  - plsc.load_gather(x2v, [ich, icl]))
            d2 = dx * dx + dy * dy + dz * dz
            rows = grp * 16 + lax.broadcasted_iota(jnp.int32, (16,), 0)
            for k, v in enumerate((dx, dy, dz, d2)):
                lanes = jnp.full((16,), k, jnp.int32)
                plsc.store_scatter(bufx, [rows, lanes], v)

        pltpu.sync_copy(bufx, cd_hbm.at[pl.ds(ebase + j * CHUNK, CHUNK), :])
        return carry

    lax.fori_loop(0, nch, step, 0)


def _sc_gather_b_body(nch, d, hc_hbm, col_hbm, dep_hbm, g_hbm, idxc, bufa,
                      depb, s1):
    pltpu.sync_copy(dep_hbm.at[pl.ds(0, 8), :], depb)
    cid = lax.axis_index("c")
    sid = lax.axis_index("s")
    wid = sid * NCORE + cid
    ebase = wid * nch * CHUNK
    pltpu.sync_copy(col_hbm.at[wid], idxc)

    def step(j, carry):
        pltpu.async_copy(hc_hbm.at[idxc.at[j]], bufa, s1).wait()
        pltpu.sync_copy(bufa, g_hbm.at[pl.ds(ebase + j * CHUNK, CHUNK), :])
        return carry

    lax.fori_loop(0, nch, step, 0)


def _sc_gather(hr, hc, x0, x1, x2, row2d, col2d):
    n, d = hr.shape
    nch = row2d.shape[1]
    e = TILES * nch * CHUNK
    mesh = plsc.VectorSubcoreMesh(core_axis_name="c", subcore_axis_name="s")
    kern_cd = pl.kernel(
        functools.partial(_sc_cd_body, nch),
        mesh=mesh,
        compiler_params=pltpu.CompilerParams(needs_layout_passes=False),
        out_type=jax.ShapeDtypeStruct((e, XP), F32),
        scratch_types=[
            pltpu.VMEM((nch, CHUNK), jnp.int32),
            pltpu.VMEM((nch, CHUNK), jnp.int32),
            pltpu.VMEM((CHUNK, XP), F32),
            pltpu.VMEM(((n + 127) // 128, 128), F32),
            pltpu.VMEM(((n + 127) // 128, 128), F32),
            pltpu.VMEM(((n + 127) // 128, 128), F32),
        ],
    )
    def make_b(depw):
        return pl.kernel(
            functools.partial(_sc_gather_b_body, nch, d),
            mesh=mesh,
            compiler_params=pltpu.CompilerParams(needs_layout_passes=False),
            out_type=jax.ShapeDtypeStruct((e, d), F32),
            scratch_types=[
                pltpu.VMEM((nch, CHUNK), jnp.int32),
                pltpu.VMEM((CHUNK, d), F32),
                pltpu.VMEM((8, depw), F32),
                pltpu.SemaphoreType.DMA,
            ],
        )

    cd = kern_cd(x0, x1, x2, row2d, col2d)
    g1 = make_b(XP)(hr, row2d, cd)
    g2 = make_b(d)(hc, col2d, g1)
    return g1, g2, cd


def _edge_body(g_ref, g2_ref, cd_ref, ea_ref, weat_ref, be1_ref, wd_ref,
               we2t_ref, be2_ref, wc1t_ref, bc1_ref, wc2_ref, mij_ref,
               wdiff_ref):
    cd = cd_ref[...]
    d2 = cd[:, 3:4]
    pre = (g_ref[...] + g2_ref[...] + d2 * wd_ref[...] + be1_ref[...]
           + jnp.dot(ea_ref[...], weat_ref[...], preferred_element_type=F32))
    m = _silu(pre)
    mij = _silu(jnp.dot(m, we2t_ref[...], preferred_element_type=F32)
                + be2_ref[...])
    mij_ref[...] = mij
    p = _silu(jnp.dot(mij, wc1t_ref[...], preferred_element_type=F32)
              + bc1_ref[...])
    cw = jnp.tanh(jnp.sum(p * wc2_ref[...], axis=1, keepdims=True))
    inv_dist = lax.rsqrt(d2 + 1e-8)
    lane = lax.broadcasted_iota(jnp.int32, (1, XP), 1)
    ones3 = jnp.where(lane == 3, 1.0, 0.0).astype(F32)
    cd3 = jnp.where(lane < 3, cd, 0.0)
    wdiff_ref[...] = cd3 * (cw * inv_dist) + ones3


def _edge_mlp(g, g2, cd, ea, weat, be1, wd, we2t, be2, wc1t, bc1, wc2, be):
    e, d = g.shape
    de = ea.shape[1]
    grid = e // be
    full = lambda r, c: pl.BlockSpec((r, c), lambda i: (0, 0))
    return pl.pallas_call(
        _edge_body,
        grid=(grid,),
        in_specs=[
            pl.BlockSpec((be, d), lambda i: (i, 0)),
            pl.BlockSpec((be, d), lambda i: (i, 0)),
            pl.BlockSpec((be, XP), lambda i: (i, 0)),
            pl.BlockSpec((be, de), lambda i: (i, 0)),
            full(de, d), full(1, d), full(1, d),
            full(d, d), full(1, d),
            full(d, d), full(1, d), full(1, d),
        ],
        out_specs=[
            pl.BlockSpec((be, d), lambda i: (i, 0)),
            pl.BlockSpec((be, XP), lambda i: (i, 0)),
        ],
        out_shape=[
            jax.ShapeDtypeStruct((e, d), F32),
            jax.ShapeDtypeStruct((e, XP), F32),
        ],
    )(g, g2, cd, ea, weat, be1, wd, we2t, be2, wc1t, bc1, wc2)


def _sc_scatter_body(nch, npt, d, mij_hbm, wd_hbm, row_hbm, zmi_hbm,
                     mi_out, xu_out, idx, bufm, bufw, bufw128, smi):
    cid = lax.axis_index("c")
    sid = lax.axis_index("s")
    wid = sid * NCORE + cid
    ebase = wid * nch * CHUNK
    nb = sid * npt
    pltpu.sync_copy(zmi_hbm, smi.at[pl.ds(nb, npt), :])
    pltpu.sync_copy(row_hbm.at[wid], idx)
    plsc.subcore_barrier()

    def step_a(j, carry):
        pltpu.sync_copy(mij_hbm.at[pl.ds(ebase + j * CHUNK, CHUNK), :], bufm)
        pltpu.sync_copy(bufm, smi.at[idx.at[j]], add=True)
        return carry

    lax.fori_loop(0, nch, step_a, 0)
    plsc.subcore_barrier()
    pltpu.sync_copy(smi.at[pl.ds(nb, npt), :], mi_out.at[cid, pl.ds(nb, npt), :])
    plsc.subcore_barrier()

    pltpu.sync_copy(zmi_hbm, smi.at[pl.ds(nb, npt), :])
    pltpu.sync_copy(zmi_hbm.at[pl.ds(0, CHUNK), :], bufw128)
    plsc.subcore_barrier()

    def step_b(j, carry):
        pltpu.sync_copy(wd_hbm.at[pl.ds(ebase + j * CHUNK, CHUNK), :], bufw)

        def putrow(r, c2):
            bufw128[r, pl.ds(0, XP)] = bufw[r, :]
            return c2

        lax.fori_loop(0, CHUNK, putrow, 0)
        pltpu.sync_copy(bufw128, smi.at[idx.at[j]], add=True)
        return carry

    lax.fori_loop(0, nch, step_b, 0)
    plsc.subcore_barrier()
    pltpu.sync_copy(smi.at[pl.ds(nb, npt), :], xu_out.at[cid, pl.ds(nb, npt), :])


def _sc_scatter(mij, wdiff, row2d, n):
    e, d = mij.shape
    nch = row2d.shape[1]
    npad = ((n + 8 * NSUB - 1) // (8 * NSUB)) * 8 * NSUB
    npt = npad // NSUB
    zmi = jnp.zeros((npt, d), F32)
    mesh = plsc.VectorSubcoreMesh(core_axis_name="c", subcore_axis_name="s")
    kern = pl.kernel(
        functools.partial(_sc_scatter_body, nch, npt, d),
        mesh=mesh,
        compiler_params=pltpu.CompilerParams(needs_layout_passes=False),
        out_type=[
            jax.ShapeDtypeStruct((NCORE, npad, d), F32),
            jax.ShapeDtypeStruct((NCORE, npad, d), F32),
        ],
        scratch_types=[
            pltpu.VMEM((nch, CHUNK), jnp.int32),
            pltpu.VMEM((CHUNK, d), F32),
            pltpu.VMEM((CHUNK, XP), F32),
            pltpu.VMEM((CHUNK, d), F32),
            pltpu.VMEM_SHARED((npad, d), F32),
        ],
    )
    mi2, xu2 = kern(mij, wdiff, row2d, zmi)
    return mi2[:, :n, :], xu2[:, :n, :]


def _node_body(h_ref, mi_ref, xu_ref, xp_ref, wht_ref, wmt_ref, bn1_ref,
               wn2t_ref, bn2_ref, lng_ref, lnb_ref, ho_ref, xo_ref):
    h = h_ref[...]
    mi = mi_ref[0] + mi_ref[1]
    t = _silu(jnp.dot(h, wht_ref[...], preferred_element_type=F32)
              + jnp.dot(mi, wmt_ref[...], preferred_element_type=F32)
              + bn1_ref[...])
    hn = h + jnp.dot(t, wn2t_ref[...], preferred_element_type=F32) + bn2_ref[...]
    mu = jnp.mean(hn, axis=1, keepdims=True)
    var = jnp.mean((hn - mu) * (hn - mu), axis=1, keepdims=True)
    ho_ref[...] = (hn - mu) * lax.rsqrt(var + 1e-5) * lng_ref[...] + lnb_ref[...]
    xu = xu_ref[0, :, :XP] + xu_ref[1, :, :XP]
    deg = jnp.maximum(xu[:, 3:4], 1.0)
    xo_ref[...] = xp_ref[...] + xu / deg


def _node_update(h, mi2, xu2, xpad, wht, wmt, bn1, wn2t, bn2, lng, lnb, bn):
    n, d = h.shape
    grid = n // bn
    full = lambda r, c: pl.BlockSpec((r, c), lambda i: (0, 0))
    return pl.pallas_call(
        _node_body,
        grid=(grid,),
        in_specs=[
            pl.BlockSpec((bn, d), lambda i: (i, 0)),
            pl.BlockSpec((NCORE, bn, d), lambda i: (0, i, 0)),
            pl.BlockSpec((NCORE, bn, d), lambda i: (0, i, 0)),
            pl.BlockSpec((bn, XP), lambda i: (i, 0)),
            full(d, d), full(d, d), full(1, d),
            full(d, d), full(1, d), full(1, d), full(1, d),
        ],
        out_specs=[
            pl.BlockSpec((bn, d), lambda i: (i, 0)),
            pl.BlockSpec((bn, XP), lambda i: (i, 0)),
        ],
        out_shape=[
            jax.ShapeDtypeStruct((n, d), F32),
            jax.ShapeDtypeStruct((n, XP), F32),
        ],
    )(h, mi2, xu2, xpad, wht, wmt, bn1, wn2t, bn2, lng, lnb)


def kernel(h, x, edge_attr, We1, be1, We2, be2, Wc1, bc1, Wc2, Wn1, bn1,
           Wn2, bn2, ln_g, ln_b, edge_index):
    n, d = h.shape
    e = edge_attr.shape[0]
    row = edge_index[0].astype(jnp.int32)
    col = edge_index[1].astype(jnp.int32)
    row2d = row.reshape(TILES, e // (TILES * CHUNK), CHUNK)
    col2d = col.reshape(TILES, e // (TILES * CHUNK), CHUNK)
    xpad = jnp.pad(x.astype(F32), ((0, 0), (0, XP - 3)))

    wrt = We1[:, :d].T
    wct = We1[:, d:2 * d].T
    wd = We1[:, 2 * d][None, :]
    weat = We1[:, 2 * d + 1:].T
    hr, hc = _prep(h, wrt, wct, bn=2000)

    npad128 = ((n + 127) // 128) * 128
    xc = jnp.pad(x.astype(F32), ((0, npad128 - n), (0, 0)))
    g1, g2, cd = _sc_gather(hr, hc,
                            xc[:, 0].reshape(npad128 // 128, 128),
                            xc[:, 1].reshape(npad128 // 128, 128),
                            xc[:, 2].reshape(npad128 // 128, 128), row2d, col2d)

    mij, wdiff = _edge_mlp(
        g1, g2, cd, edge_attr, weat, be1[None, :], wd,
        We2.T, be2[None, :], Wc1.T, bc1[None, :], Wc2, be=2560)

    mi2, xu2 = _sc_scatter(mij, wdiff, row2d, n)

    h_out, xo = _node_update(
        h, mi2, xu2, xpad, Wn1[:, :d].T, Wn1[:, d:].T, bn1[None, :],
        Wn2.T, bn2[None, :], ln_g[None, :], ln_b[None, :], bn=2000)
    return h_out, xo[:, :3]


# speedup vs baseline: 4.5914x; 1.2199x over previous
import functools

import jax
import jax.numpy as jnp
from jax import lax
from jax.experimental import pallas as pl
from jax.experimental.pallas import tpu as pltpu
from jax.experimental.pallas import tpu_sc as plsc

F32 = jnp.float32
XP = 16
CHUNK = 80
NCORE = 2
NSUB = 16
TILES = NCORE * NSUB


def _silu(v):
    return v * jax.nn.sigmoid(v)


def _prep_body(h_ref, wrt_ref, wct_ref, hr_ref, hc_ref):
    h = h_ref[...]
    hr_ref[...] = jnp.dot(h, wrt_ref[...], preferred_element_type=F32)
    hc_ref[...] = jnp.dot(h, wct_ref[...], preferred_element_type=F32)


def _prep(h, wrt, wct, bn):
    n, d = h.shape
    grid = n // bn
    return pl.pallas_call(
        _prep_body,
        grid=(grid,),
        in_specs=[
            pl.BlockSpec((bn, d), lambda i: (i, 0)),
            pl.BlockSpec((d, d), lambda i: (0, 0)),
            pl.BlockSpec((d, d), lambda i: (0, 0)),
        ],
        out_specs=[
            pl.BlockSpec((bn, d), lambda i: (i, 0)),
            pl.BlockSpec((bn, d), lambda i: (i, 0)),
        ],
        out_shape=[
            jax.ShapeDtypeStruct((n, d), F32),
            jax.ShapeDtypeStruct((n, d), F32),
        ],
    )(h, wrt, wct)


def _sc_cd_body(nch, x0_hbm, x1_hbm, x2_hbm, row_hbm, col_hbm, cd_hbm,
                idxr, idxc, bufx, x0v, x1v, x2v):
    cid = lax.axis_index("c")
    sid = lax.axis_index("s")
    wid = sid * NCORE + cid
    ebase = wid * nch * CHUNK
    pltpu.sync_copy(row_hbm.at[wid], idxr)
    pltpu.sync_copy(col_hbm.at[wid], idxc)
    pltpu.sync_copy(x0_hbm, x0v)
    pltpu.sync_copy(x1_hbm, x1v)
    pltpu.sync_copy(x2_hbm, x2v)

    def step(j, carry):
        for grp in range(CHUNK // 16):
            sl = pl.ds(grp * 16, 16)
            ir = idxr[j, sl]
            ic = idxc[j, sl]
            irh, irl = ir >> 7, ir & 127
            ich, icl = ic >> 7, ic & 127
            dx = (plsc.load_gather(x0v, [irh, irl])
                  - plsc.load_gather(x0v, [ich, icl]))
            dy = (plsc.load_gather(x1v, [irh, irl])
                  - plsc.load_gather(x1v, [ich, icl]))
            dz = (plsc.load_gather(x2v, [irh, irl])
                  - plsc.load_gather(x2v, [ich, icl]))
            d2 = dx * dx + dy * dy + dz * dz
            rows = grp * 16 + lax.broadcasted_iota(jnp.int32, (16,), 0)
            for k, v in enumerate((dx, dy, dz, d2)):
                lanes = jnp.full((16,), k, jnp.int32)
                plsc.store_scatter(bufx, [rows, lanes], v)

        pltpu.sync_copy(bufx, cd_hbm.at[pl.ds(ebase + j * CHUNK, CHUNK), :])
        return carry

    lax.fori_loop(0, nch, step, 0)


def _sc_gather_b_body(nch, d, hc_hbm, col_hbm, dep_hbm, g_hbm, idxc, buf0,
                      buf1, depb, s0, s1, w0, w1):
    pltpu.sync_copy(dep_hbm.at[pl.ds(0, 8), :], depb)
    cid = lax.axis_index("c")
    sid = lax.axis_index("s")
    wid = sid * NCORE + cid
    ebase = wid * nch * CHUNK
    pltpu.sync_copy(col_hbm.at[wid], idxc)

    bufs = (buf0, buf1)
    gsem = (s0, s1)
    wsem = (w0, w1)

    def gout(j):
        return g_hbm.at[pl.ds(ebase + j * CHUNK, CHUNK), :]

    pltpu.async_copy(hc_hbm.at[idxc.at[0]], buf0, s0)
    pltpu.async_copy(hc_hbm.at[idxc.at[1]], buf1, s1)

    def pair(jj, carry):
        j0 = jj * 2
        for b in range(2):
            j = j0 + b
            pltpu.make_async_copy(hc_hbm.at[idxc.at[j]], bufs[b], gsem[b]).wait()
            pltpu.async_copy(bufs[b], gout(j), wsem[b])
        for b in range(2):
            j = j0 + b
            pltpu.make_async_copy(bufs[b], gout(j), wsem[b]).wait()
            nxt = j + 2

            @pl.when(nxt < nch)
            def _():
                pltpu.async_copy(hc_hbm.at[idxc.at[nxt]], bufs[b], gsem[b])

        return carry

    lax.fori_loop(0, nch // 2, pair, 0)
    if nch % 2:
        j = nch - 1
        pltpu.make_async_copy(hc_hbm.at[idxc.at[j]], buf0, s0).wait()
        pltpu.sync_copy(buf0, gout(j))


def _sc_gather(hr, hc, x0, x1, x2, row2d, col2d):
    n, d = hr.shape
    nch = row2d.shape[1]
    e = TILES * nch * CHUNK
    mesh = plsc.VectorSubcoreMesh(core_axis_name="c", subcore_axis_name="s")
    kern_cd = pl.kernel(
        functools.partial(_sc_cd_body, nch),
        mesh=mesh,
        compiler_params=pltpu.CompilerParams(needs_layout_passes=False),
        out_type=jax.ShapeDtypeStruct((e, XP), F32),
        scratch_types=[
            pltpu.VMEM((nch, CHUNK), jnp.int32),
            pltpu.VMEM((nch, CHUNK), jnp.int32),
            pltpu.VMEM((CHUNK, XP), F32),
            pltpu.VMEM(((n + 127) // 128, 128), F32),
            pltpu.VMEM(((n + 127) // 128, 128), F32),
            pltpu.VMEM(((n + 127) // 128, 128), F32),
        ],
    )
    def make_b(depw):
        return pl.kernel(
            functools.partial(_sc_gather_b_body, nch, d),
            mesh=mesh,
            compiler_params=pltpu.CompilerParams(needs_layout_passes=False),
            out_type=jax.ShapeDtypeStruct((e, d), F32),
            scratch_types=[
                pltpu.VMEM((nch, CHUNK), jnp.int32),
                pltpu.VMEM((CHUNK, d), F32),
                pltpu.VMEM((CHUNK, d), F32),
                pltpu.VMEM((8, depw), F32),
                pltpu.SemaphoreType.DMA,
                pltpu.SemaphoreType.DMA,
                pltpu.SemaphoreType.DMA,
                pltpu.SemaphoreType.DMA,
            ],
        )

    cd = kern_cd(x0, x1, x2, row2d, col2d)
    g1 = make_b(XP)(hr, row2d, cd)
    g2 = make_b(d)(hc, col2d, g1)
    return g1, g2, cd


def _edge_body(g_ref, g2_ref, cd_ref, ea_ref, weat_ref, be1_ref, wd_ref,
               we2t_ref, be2_ref, wc1t_ref, bc1_ref, wc2_ref, mij_ref,
               wdiff_ref):
    cd = cd_ref[...]
    d2 = cd[:, 3:4]
    pre = (g_ref[...] + g2_ref[...] + d2 * wd_ref[...] + be1_ref[...]
           + jnp.dot(ea_ref[...], weat_ref[...], preferred_element_type=F32))
    m = _silu(pre)
    mij = _silu(jnp.dot(m, we2t_ref[...], preferred_element_type=F32)
                + be2_ref[...])
    mij_ref[...] = mij
    p = _silu(jnp.dot(mij, wc1t_ref[...], preferred_element_type=F32)
              + bc1_ref[...])
    cw = jnp.tanh(jnp.sum(p * wc2_ref[...], axis=1, keepdims=True))
    inv_dist = lax.rsqrt(d2 + 1e-8)
    lane = lax.broadcasted_iota(jnp.int32, (1, XP), 1)
    ones3 = jnp.where(lane == 3, 1.0, 0.0).astype(F32)
    cd3 = jnp.where(lane < 3, cd, 0.0)
    wdiff_ref[...] = cd3 * (cw * inv_dist) + ones3


def _edge_mlp(g, g2, cd, ea, weat, be1, wd, we2t, be2, wc1t, bc1, wc2, be):
    e, d = g.shape
    de = ea.shape[1]
    grid = e // be
    full = lambda r, c: pl.BlockSpec((r, c), lambda i: (0, 0))
    return pl.pallas_call(
        _edge_body,
        grid=(grid,),
        in_specs=[
            pl.BlockSpec((be, d), lambda i: (i, 0)),
            pl.BlockSpec((be, d), lambda i: (i, 0)),
            pl.BlockSpec((be, XP), lambda i: (i, 0)),
            pl.BlockSpec((be, de), lambda i: (i, 0)),
            full(de, d), full(1, d), full(1, d),
            full(d, d), full(1, d),
            full(d, d), full(1, d), full(1, d),
        ],
        out_specs=[
            pl.BlockSpec((be, d), lambda i: (i, 0)),
            pl.BlockSpec((be, XP), lambda i: (i, 0)),
        ],
        out_shape=[
            jax.ShapeDtypeStruct((e, d), F32),
            jax.ShapeDtypeStruct((e, XP), F32),
        ],
    )(g, g2, cd, ea, weat, be1, wd, we2t, be2, wc1t, bc1, wc2)


def _sc_scatter_body(nch, npt, d, mij_hbm, wd_hbm, row_hbm, zmi_hbm,
                     mi_out, xu_out, idx, bufm, bufw, bufw128, smi, sa0, sa1):
    cid = lax.axis_index("c")
    sid = lax.axis_index("s")
    wid = sid * NCORE + cid
    ebase = wid * nch * CHUNK
    nb = sid * npt
    pltpu.sync_copy(zmi_hbm, smi.at[pl.ds(nb, npt), :])
    pltpu.sync_copy(row_hbm.at[wid], idx)
    plsc.subcore_barrier()

    bufs = (bufm, bufw128)
    sems = (sa0, sa1)

    def mij_in(j):
        return mij_hbm.at[pl.ds(ebase + j * CHUNK, CHUNK), :]

    pltpu.async_copy(mij_in(0), bufm, sa0)
    pltpu.async_copy(mij_in(1), bufw128, sa1)

    def pair_a(jj, carry):
        j0 = jj * 2
        for b in range(2):
            j = j0 + b
            pltpu.make_async_copy(mij_in(j), bufs[b], sems[b]).wait()
            pltpu.sync_copy(bufs[b], smi.at[idx.at[j]], add=True)
            nxt = j + 2

            @pl.when(nxt < nch)
            def _():
                pltpu.async_copy(mij_in(nxt), bufs[b], sems[b])

        return carry

    lax.fori_loop(0, nch // 2, pair_a, 0)
    if nch % 2:
        j = nch - 1
        pltpu.make_async_copy(mij_in(j), bufm, sa0).wait()
        pltpu.sync_copy(bufm, smi.at[idx.at[j]], add=True)
    plsc.subcore_barrier()
    pltpu.sync_copy(smi.at[pl.ds(nb, npt), :], mi_out.at[cid, pl.ds(nb, npt), :])
    plsc.subcore_barrier()

    pltpu.sync_copy(zmi_hbm, smi.at[pl.ds(nb, npt), :])
    pltpu.sync_copy(zmi_hbm.at[pl.ds(0, CHUNK), :], bufm)
    plsc.subcore_barrier()

    def wd_in(j):
        return wd_hbm.at[pl.ds(ebase + j * CHUNK, CHUNK), :]

    pltpu.async_copy(wd_in(0), bufw, sa0)

    def step_b(j, carry):
        pltpu.make_async_copy(wd_in(j), bufw, sa0).wait()

        def putrow(r, c2):
            bufm[r, pl.ds(0, XP)] = bufw[r, :]
            return c2

        lax.fori_loop(0, CHUNK, putrow, 0)

        @pl.when(j + 1 < nch)
        def _():
            pltpu.async_copy(wd_in(j + 1), bufw, sa0)

        pltpu.sync_copy(bufm, smi.at[idx.at[j]], add=True)
        return carry

    lax.fori_loop(0, nch, step_b, 0)
    plsc.subcore_barrier()
    pltpu.sync_copy(smi.at[pl.ds(nb, npt), :], xu_out.at[cid, pl.ds(nb, npt), :])


def _sc_scatter(mij, wdiff, row2d, n):
    e, d = mij.shape
    nch = row2d.shape[1]
    npad = ((n + 8 * NSUB - 1) // (8 * NSUB)) * 8 * NSUB
    npt = npad // NSUB
    zmi = jnp.zeros((npt, d), F32)
    mesh = plsc.VectorSubcoreMesh(core_axis_name="c", subcore_axis_name="s")
    kern = pl.kernel(
        functools.partial(_sc_scatter_body, nch, npt, d),
        mesh=mesh,
        compiler_params=pltpu.CompilerParams(needs_layout_passes=False),
        out_type=[
            jax.ShapeDtypeStruct((NCORE, npad, d), F32),
            jax.ShapeDtypeStruct((NCORE, npad, d), F32),
        ],
        scratch_types=[
            pltpu.VMEM((nch, CHUNK), jnp.int32),
            pltpu.VMEM((CHUNK, d), F32),
            pltpu.VMEM((CHUNK, XP), F32),
            pltpu.VMEM((CHUNK, d), F32),
            pltpu.VMEM_SHARED((npad, d), F32),
            pltpu.SemaphoreType.DMA,
            pltpu.SemaphoreType.DMA,
        ],
    )
    mi2, xu2 = kern(mij, wdiff, row2d, zmi)
    return mi2[:, :n, :], xu2[:, :n, :]


def _node_body(h_ref, mi_ref, xu_ref, xp_ref, wht_ref, wmt_ref, bn1_ref,
               wn2t_ref, bn2_ref, lng_ref, lnb_ref, ho_ref, xo_ref):
    h = h_ref[...]
    mi = mi_ref[0] + mi_ref[1]
    t = _silu(jnp.dot(h, wht_ref[...], preferred_element_type=F32)
              + jnp.dot(mi, wmt_ref[...], preferred_element_type=F32)
              + bn1_ref[...])
    hn = h + jnp.dot(t, wn2t_ref[...], preferred_element_type=F32) + bn2_ref[...]
    mu = jnp.mean(hn, axis=1, keepdims=True)
    var = jnp.mean((hn - mu) * (hn - mu), axis=1, keepdims=True)
    ho_ref[...] = (hn - mu) * lax.rsqrt(var + 1e-5) * lng_ref[...] + lnb_ref[...]
    xu = xu_ref[0, :, :XP] + xu_ref[1, :, :XP]
    deg = jnp.maximum(xu[:, 3:4], 1.0)
    xo_ref[...] = xp_ref[...] + xu / deg


def _node_update(h, mi2, xu2, xpad, wht, wmt, bn1, wn2t, bn2, lng, lnb, bn):
    n, d = h.shape
    grid = n // bn
    full = lambda r, c: pl.BlockSpec((r, c), lambda i: (0, 0))
    return pl.pallas_call(
        _node_body,
        grid=(grid,),
        in_specs=[
            pl.BlockSpec((bn, d), lambda i: (i, 0)),
            pl.BlockSpec((NCORE, bn, d), lambda i: (0, i, 0)),
            pl.BlockSpec((NCORE, bn, d), lambda i: (0, i, 0)),
            pl.BlockSpec((bn, XP), lambda i: (i, 0)),
            full(d, d), full(d, d), full(1, d),
            full(d, d), full(1, d), full(1, d), full(1, d),
        ],
        out_specs=[
            pl.BlockSpec((bn, d), lambda i: (i, 0)),
            pl.BlockSpec((bn, XP), lambda i: (i, 0)),
        ],
        out_shape=[
            jax.ShapeDtypeStruct((n, d), F32),
            jax.ShapeDtypeStruct((n, XP), F32),
        ],
    )(h, mi2, xu2, xpad, wht, wmt, bn1, wn2t, bn2, lng, lnb)


def kernel(h, x, edge_attr, We1, be1, We2, be2, Wc1, bc1, Wc2, Wn1, bn1,
           Wn2, bn2, ln_g, ln_b, edge_index):
    n, d = h.shape
    e = edge_attr.shape[0]
    row = edge_index[0].astype(jnp.int32)
    col = edge_index[1].astype(jnp.int32)
    row2d = row.reshape(TILES, e // (TILES * CHUNK), CHUNK)
    col2d = col.reshape(TILES, e // (TILES * CHUNK), CHUNK)
    xpad = jnp.pad(x.astype(F32), ((0, 0), (0, XP - 3)))

    wrt = We1[:, :d].T
    wct = We1[:, d:2 * d].T
    wd = We1[:, 2 * d][None, :]
    weat = We1[:, 2 * d + 1:].T
    hr, hc = _prep(h, wrt, wct, bn=2000)

    npad128 = ((n + 127) // 128) * 128
    xc = jnp.pad(x.astype(F32), ((0, npad128 - n), (0, 0)))
    g1, g2, cd = _sc_gather(hr, hc,
                            xc[:, 0].reshape(npad128 // 128, 128),
                            xc[:, 1].reshape(npad128 // 128, 128),
                            xc[:, 2].reshape(npad128 // 128, 128), row2d, col2d)

    mij, wdiff = _edge_mlp(
        g1, g2, cd, edge_attr, weat, be1[None, :], wd,
        We2.T, be2[None, :], Wc1.T, bc1[None, :], Wc2, be=2560)

    mi2, xu2 = _sc_scatter(mij, wdiff, row2d, n)

    h_out, xo = _node_update(
        h, mi2, xu2, xpad, Wn1[:, :d].T, Wn1[:, d:].T, bn1[None, :],
        Wn2.T, bn2[None, :], ln_g[None, :], ln_b[None, :], bn=2000)
    return h_out, xo[:, :3]
